# baseline (speedup 1.0000x reference)
import functools

import jax
import jax.numpy as jnp
from jax.experimental import pallas as pl
import jax.experimental.pallas.tpu as pltpu

_TOP_K = 8
_NEG = -1e30
_RBLK = 128


def _body(facts_ref, beta_ref, alT_ref, rs_ref, W_ref, gamma_ref, lnb_ref,
          out_ref, act_ref, pre_ref, logf_ref, log1mf_ref):
    i = pl.program_id(0)
    nb = pl.num_programs(0)
    facts = facts_ref[...]

    @pl.when(i == 0)
    def _():
        logf_ref[...] = jnp.log(jnp.maximum(facts, 1e-30))
        log1mf_ref[...] = jnp.log(jnp.maximum(1.0 - facts, 1e-30))

    mask = (beta_ref[...] > 0.0).astype(jnp.float32)

    dn = (((1,), (1,)), ((), ()))
    mm = functools.partial(jax.lax.dot_general, dimension_numbers=dn,
                           preferred_element_type=jnp.float32,
                           precision=jax.lax.Precision.HIGHEST)

    s_sum = mm(facts, mask)
    and_agg = jnp.exp(mm(logf_ref[...], mask))
    or_agg = 1.0 - jnp.exp(mm(log1mf_ref[...], mask))
    cnt = jnp.sum(mask, axis=1)[None, :] + 1e-08
    k_of_n = s_sum / cnt

    sl = pl.ds(i * _RBLK, _RBLK)
    w = jax.nn.softmax(alT_ref[:, sl], axis=0)
    mixed = (w[0][None, :] * and_agg + w[1][None, :] * or_agg
             + w[2][None, :] * k_of_n + w[3][None, :] * (1.0 - k_of_n))
    act_ref[:, sl] = mixed * jax.nn.sigmoid(rs_ref[:, sl])
    pre_ref[:, sl] = mm(facts, W_ref[...])

    @pl.when(i == nb - 1)
    def _():
        act = act_ref[...]
        iota = jax.lax.broadcasted_iota(jnp.int32, act.shape, 1)
        a = act
        gate = jnp.zeros_like(act)
        for _ in range(_TOP_K):
            m = jnp.max(a, axis=1, keepdims=True)
            idx = jnp.min(jnp.where(a == m, iota, act.shape[1]), axis=1,
                          keepdims=True)
            sel = iota == idx
            gate = jnp.where(sel, 1.0, gate)
            a = jnp.where(sel, _NEG, a)

        pre = pre_ref[...] + act * gate
        mu = jnp.mean(pre, axis=1, keepdims=True)
        var = jnp.mean((pre - mu) ** 2, axis=1, keepdims=True)
        out_ref[...] = ((pre - mu) * jax.lax.rsqrt(var + 1e-05)
                        * gamma_ref[...] + lnb_ref[...])


def kernel(facts, beta, aggregator_logits, rule_strength_raw, W, gamma,
           ln_beta):
    B, D = facts.shape
    R, _ = beta.shape
    nb = R // _RBLK
    return pl.pallas_call(
        _body,
        grid=(nb,),
        in_specs=[
            pl.BlockSpec((B, D), lambda i: (0, 0)),
            pl.BlockSpec((_RBLK, D), lambda i: (i, 0)),
            pl.BlockSpec((4, R), lambda i: (0, 0)),
            pl.BlockSpec((1, R), lambda i: (0, 0)),
            pl.BlockSpec((_RBLK, D), lambda i: (i, 0)),
            pl.BlockSpec((1, R), lambda i: (0, 0)),
            pl.BlockSpec((1, R), lambda i: (0, 0)),
        ],
        out_specs=pl.BlockSpec((B, R), lambda i: (0, 0)),
        out_shape=jax.ShapeDtypeStruct((B, R), jnp.float32),
        scratch_shapes=[
            pltpu.VMEM((B, R), jnp.float32),
            pltpu.VMEM((B, R), jnp.float32),
            pltpu.VMEM((B, D), jnp.float32),
            pltpu.VMEM((B, D), jnp.float32),
        ],
    )(facts, beta, aggregator_logits.T, rule_strength_raw[None, :], W,
      gamma[None, :], ln_beta[None, :])


# speedup vs baseline: 1.7465x; 1.7465x over previous
import functools

import jax
import jax.numpy as jnp
from jax.experimental import pallas as pl

_TOP_K = 8
_NEG = -1e30


def _body(facts_ref, beta_ref, alT_ref, rs_ref, W_ref, gamma_ref, lnb_ref,
          out_ref):
    facts = facts_ref[...]
    B = facts.shape[0]
    beta = beta_ref[...]
    mask = jnp.where(beta > 0.0, 1.0, 0.0)
    mask_bf = mask.astype(jnp.bfloat16)

    dn = (((1,), (1,)), ((), ()))

    log_f = jnp.log(jnp.maximum(facts, 1e-30))
    log_1mf = jnp.log(jnp.maximum(1.0 - facts, 1e-30))
    logs = jnp.concatenate([log_f, log_1mf], axis=0).astype(jnp.bfloat16)
    prods = jnp.exp(jax.lax.dot_general(
        logs, mask_bf, dimension_numbers=dn,
        preferred_element_type=jnp.float32))
    and_agg = prods[:B]
    or_agg = 1.0 - prods[B:]

    mm = functools.partial(jax.lax.dot_general, dimension_numbers=dn,
                           preferred_element_type=jnp.float32,
                           precision=jax.lax.Precision.HIGHEST)
    s_sum = mm(facts, mask)
    cnt = jnp.sum(mask, axis=1)[None, :] + 1e-08
    k_of_n = s_sum / cnt

    w = jax.nn.softmax(alT_ref[...], axis=0)
    mixed = (w[0][None, :] * and_agg + w[1][None, :] * or_agg
             + w[2][None, :] * k_of_n + w[3][None, :] * (1.0 - k_of_n))
    act = mixed * jax.nn.sigmoid(rs_ref[...])

    iota = jax.lax.broadcasted_iota(jnp.int32, act.shape, 1)
    a = act
    gate = jnp.zeros_like(act)
    for _ in range(_TOP_K):
        m = jnp.max(a, axis=1, keepdims=True)
        idx = jnp.min(jnp.where(a == m, iota, act.shape[1]), axis=1,
                      keepdims=True)
        sel = iota == idx
        gate = jnp.where(sel, 1.0, gate)
        a = jnp.where(sel, _NEG, a)

    pre = mm(facts, W_ref[...]) + act * gate
    mu = jnp.mean(pre, axis=1, keepdims=True)
    var = jnp.mean(pre * pre, axis=1, keepdims=True) - mu * mu
    out_ref[...] = ((pre - mu) * jax.lax.rsqrt(var + 1e-05)
                    * gamma_ref[...] + lnb_ref[...])


def kernel(facts, beta, aggregator_logits, rule_strength_raw, W, gamma,
           ln_beta):
    B, _ = facts.shape
    R, _ = beta.shape
    return pl.pallas_call(
        _body,
        out_shape=jax.ShapeDtypeStruct((B, R), jnp.float32),
    )(facts, beta, aggregator_logits.T, rule_strength_raw[None, :], W,
      gamma[None, :], ln_beta[None, :])
